# baseline (speedup 1.0000x reference)
import functools

import jax
import jax.numpy as jnp
from jax import lax
from jax.experimental import pallas as pl
from jax.experimental.pallas import tpu as pltpu
from jax.experimental.pallas import tpu_sc as plsc

N = 10000
E = 320000
D = 128
G = 64
DOUT = 10

NC = 2
NS = 16
NW = NC * NS
CH = 80
K = 126
KP = K + 2
K2 = 252
KP2 = K2 + 4
NPAD = 10112
RPT = NPAD // NS
DUMMY = N + 8

HREAL = 5120
HN = 5248
HRPT = HN // NS
LOCDUMMY = HREAL + 8

BN = 2560
NBLK = 4

_mesh = plsc.VectorSubcoreMesh(core_axis_name="c", subcore_axis_name="s")


@functools.partial(
    pl.kernel,
    out_type=jax.ShapeDtypeStruct((NC * NPAD,), jnp.float32),
    mesh=_mesh,
    scratch_types=[
        pltpu.VMEM((KP, CH), jnp.int32),
        pltpu.VMEM((CH,), jnp.float32),
        pltpu.VMEM((RPT + 8, ), jnp.float32),
        pltpu.VMEM_SHARED((NPAD,), jnp.float32),
    ],
)
def _sc_deg(dst_hbm, out_hbm, dst_v, ones_v, zb_v, deg_sh):
    c = lax.axis_index("c")
    s = lax.axis_index("s")
    wid = s * NC + c
    pltpu.sync_copy(dst_hbm.at[wid], dst_v)

    z16 = jnp.zeros((16,), jnp.float32)
    ones16 = jnp.ones((16,), jnp.float32)

    def _fill(i, carry):
        ones_v[pl.ds(i * 16, 16)] = ones16
        return carry

    lax.fori_loop(0, CH // 16, _fill, 0)

    def _zero(i, carry):
        zb_v[pl.ds(i * 16, 16)] = z16
        return carry

    lax.fori_loop(0, (RPT + 8) // 16, _zero, 0)

    base = s * RPT
    pltpu.sync_copy(zb_v.at[pl.ds(0, RPT)], deg_sh.at[pl.ds(base, RPT)])
    plsc.subcore_barrier()

    def _count(j, carry):
        pltpu.sync_copy(ones_v, deg_sh.at[dst_v.at[j]], add=True)
        return carry

    lax.fori_loop(0, KP, _count, 0)
    plsc.subcore_barrier()
    pltpu.sync_copy(deg_sh.at[pl.ds(base, RPT)], zb_v.at[pl.ds(0, RPT)])
    pltpu.sync_copy(zb_v.at[pl.ds(0, RPT)],
                    out_hbm.at[pl.ds(c * NPAD + base, RPT)])


@functools.partial(
    pl.kernel,
    out_type=jax.ShapeDtypeStruct((NC, HN, D), jnp.float32),
    mesh=_mesh,
    scratch_types=[
        pltpu.VMEM((KP2, CH), jnp.int32),
        pltpu.VMEM((4, CH), jnp.int32),
        pltpu.VMEM((4, CH), jnp.int32),
        pltpu.VMEM((4, CH, D), jnp.float32),
        pltpu.VMEM_SHARED((HN, D), jnp.float32),
        (pltpu.SemaphoreType.DMA,) * 4,
        (pltpu.SemaphoreType.DMA,) * 4,
    ],
)
def _sc_scatter(yh, combo_hbm, out_hbm, combo_v, srcst, dstst, rows, acc_sh,
                gsem, ssem):
    c = lax.axis_index("c")
    s = lax.axis_index("s")
    pltpu.sync_copy(combo_hbm.at[s], combo_v)
    off = c * HREAL

    def _unpack(j, b):
        for i in range(CH // 16):
            v = combo_v[j, pl.ds(i * 16, 16)]
            srcst[b, pl.ds(i * 16, 16)] = v & 0xFFFF
            loc = (v >> 16) - off
            bad = (loc < 0) | (loc >= HREAL)
            dstst[b, pl.ds(i * 16, 16)] = jnp.where(bad, HREAL + (loc & 63),
                                                    loc)

    z16 = jnp.zeros((16,), jnp.float32)

    def _zero(i, carry):
        rows[0, i // (D // 16), pl.ds((i % (D // 16)) * 16, 16)] = z16
        return carry

    lax.fori_loop(0, CH * D // 16, _zero, 0)

    base = s * HRPT
    for k in range(HRPT // CH):
        pltpu.sync_copy(rows.at[0], acc_sh.at[pl.ds(base + k * CH, CH)])
    rem = HRPT % CH
    if rem:
        pltpu.sync_copy(rows.at[0, pl.ds(0, rem)],
                        acc_sh.at[pl.ds(base + (HRPT // CH) * CH, rem)])
    plsc.subcore_barrier()

    def _gather(j, b):
        return pltpu.make_async_copy(yh.at[srcst.at[b]], rows.at[b], gsem[b])

    def _scat(b):
        return pltpu.make_async_copy(rows.at[b], acc_sh.at[dstst.at[b]],
                                     ssem[b])

    def _scat_add(b):
        return pltpu.async_copy(rows.at[b], acc_sh.at[dstst.at[b]], ssem[b],
                                add=True)

    for b in range(4):
        _unpack(b, b)
    _gather(0, 0).start()
    _gather(1, 1).start()
    _gather(2, 2).start()
    _gather(3, 3).start()
    _gather(0, 0).wait()
    _scat_add(0)
    _gather(1, 1).wait()
    _scat_add(1)
    _gather(2, 2).wait()
    _scat_add(2)
    _scat(0).wait()
    _unpack(4, 0)
    _gather(4, 0).start()
    _gather(3, 3).wait()
    _scat_add(3)
    _scat(1).wait()
    _unpack(5, 1)
    _gather(5, 1).start()

    def _body(t, carry):
        j0 = t * 4
        for b in range(4):
            j = j0 + b
            b2 = (b + 2) % 4
            _gather(j, b).wait()
            _scat_add(b)
            _scat(b2).wait()
            _unpack(j + 2, b2)
            _gather(j + 2, b2).start()
        return carry

    lax.fori_loop(1, K2 // 4, _body, 0)
    _scat((K2 - 2) % 4).wait()
    _scat((K2 - 1) % 4).wait()
    _gather(K2, K2 % 4).wait()
    _gather(K2 + 1, (K2 + 1) % 4).wait()
    plsc.subcore_barrier()

    nfull = HRPT // CH
    rem = HRPT % CH
    for k in range(nfull):
        b = k % 2
        pltpu.sync_copy(acc_sh.at[pl.ds(base + k * CH, CH)], rows.at[b])
        pltpu.make_async_copy(rows.at[b], out_hbm.at[c, pl.ds(base + k * CH, CH)],
                              gsem[b]).start()
        if k >= 1:
            pb = (k - 1) % 2
            pltpu.make_async_copy(
                rows.at[pb], out_hbm.at[c, pl.ds(base + (k - 1) * CH, CH)],
                gsem[pb]).wait()
    if rem:
        b = nfull % 2
        pltpu.sync_copy(acc_sh.at[pl.ds(base + nfull * CH, rem)],
                        rows.at[b, pl.ds(0, rem)])
        pltpu.sync_copy(rows.at[b, pl.ds(0, rem)],
                        out_hbm.at[c, pl.ds(base + nfull * CH, rem)])
    pb = (nfull - 1) % 2
    pltpu.make_async_copy(rows.at[pb],
                          out_hbm.at[c, pl.ds(base + (nfull - 1) * CH, CH)],
                          gsem[pb]).wait()


def _acc_spec():
    return pl.BlockSpec((1, BN, D), lambda i: (i // 2, i % 2, 0))


def _mm_first_body(deg_ref, x_ref, w_ref, y_ref, dinv_ref):
    degs = jnp.maximum(deg_ref[0] + deg_ref[1] + 1.0, 1.0)
    dinv = lax.rsqrt(degs)
    dinv_ref[...] = jnp.broadcast_to(dinv[None, :], (8, BN))
    y_ref[...] = dinv[:, None] * jnp.dot(
        x_ref[...], w_ref[...], preferred_element_type=jnp.float32)


def _mm_first(deg_part, x, w):
    return pl.pallas_call(
        _mm_first_body,
        grid=(NBLK,),
        in_specs=[
            pl.BlockSpec((NC, BN), lambda i: (0, i)),
            pl.BlockSpec((BN, D), lambda i: (i, 0)),
            pl.BlockSpec((D, D), lambda i: (0, 0)),
        ],
        out_specs=[
            pl.BlockSpec((BN, D), lambda i: (i, 0)),
            pl.BlockSpec((8, BN), lambda i: (0, i)),
        ],
        out_shape=[
            jax.ShapeDtypeStruct((N, D), jnp.float32),
            jax.ShapeDtypeStruct((8, NBLK * BN), jnp.float32),
        ],
    )(deg_part, x, w)


def _mm_mid_body(acc_ref, y_ref, dinv_ref, b_ref, w_ref, out_ref):
    a = acc_ref[0] + y_ref[...]
    dv = dinv_ref[0]
    h = jnp.maximum(dv[:, None] * a + b_ref[...][None, :], 0.0)
    out_ref[...] = dv[:, None] * jnp.dot(
        h, w_ref[...], preferred_element_type=jnp.float32)


def _mm_mid(acc, y, dinv, b, w):
    return pl.pallas_call(
        _mm_mid_body,
        grid=(NBLK,),
        in_specs=[
            _acc_spec(),
            pl.BlockSpec((BN, D), lambda i: (i, 0)),
            pl.BlockSpec((8, BN), lambda i: (0, i)),
            pl.BlockSpec((D,), lambda i: (0,)),
            pl.BlockSpec((D, D), lambda i: (0, 0)),
        ],
        out_specs=pl.BlockSpec((BN, D), lambda i: (i, 0)),
        out_shape=jax.ShapeDtypeStruct((N, D), jnp.float32),
    )(acc, y, dinv, b, w)


def _mm_final_body(acc_ref, y_ref, dinv_ref, b_ref, batch_ref, wl_ref, bl_ref,
                   sums_ref, counts_ref, out_ref):
    i = pl.program_id(0)

    @pl.when(i == 0)
    def _():
        sums_ref[...] = jnp.zeros_like(sums_ref)
        counts_ref[...] = jnp.zeros_like(counts_ref)

    a = acc_ref[0] + y_ref[...]
    h = dinv_ref[0][:, None] * a + b_ref[...][None, :]
    validc = (i * BN + lax.broadcasted_iota(jnp.int32, (BN, 1), 0)) < N
    h = jnp.where(validc, h, 0.0)
    validr = (i * BN + lax.broadcasted_iota(jnp.int32, (1, BN), 1)) < N
    oh = jnp.where(
        (lax.broadcasted_iota(jnp.int32, (G, BN), 0) == batch_ref[0][None, :])
        & validr, 1.0, 0.0)
    sums_ref[...] += jnp.dot(oh, h, preferred_element_type=jnp.float32)
    counts_ref[...] += jnp.sum(oh, axis=1, keepdims=True)

    @pl.when(i == NBLK - 1)
    def _():
        g = sums_ref[...] / jnp.maximum(counts_ref[...], 1.0)
        logits = jnp.dot(g, wl_ref[...],
                         preferred_element_type=jnp.float32) + bl_ref[...][None, :]
        m = jnp.max(logits, axis=1, keepdims=True)
        lse = jnp.log(jnp.sum(jnp.exp(logits - m), axis=1, keepdims=True)) + m
        out_ref[...] = logits - lse


def _mm_final(acc, y, dinv, b, batch, wl, bl):
    _, _, out = pl.pallas_call(
        _mm_final_body,
        grid=(NBLK,),
        in_specs=[
            _acc_spec(),
            pl.BlockSpec((BN, D), lambda i: (i, 0)),
            pl.BlockSpec((8, BN), lambda i: (0, i)),
            pl.BlockSpec((D,), lambda i: (0,)),
            pl.BlockSpec((8, BN), lambda i: (0, i)),
            pl.BlockSpec((D, DOUT), lambda i: (0, 0)),
            pl.BlockSpec((DOUT,), lambda i: (0,)),
        ],
        out_specs=[
            pl.BlockSpec((G, D), lambda i: (0, 0)),
            pl.BlockSpec((G, 1), lambda i: (0, 0)),
            pl.BlockSpec((G, DOUT), lambda i: (0, 0)),
        ],
        out_shape=[
            jax.ShapeDtypeStruct((G, D), jnp.float32),
            jax.ShapeDtypeStruct((G, 1), jnp.float32),
            jax.ShapeDtypeStruct((G, DOUT), jnp.float32),
        ],
    )(acc, y, dinv, b, batch, wl, bl)
    return out


def kernel(x, edge_index, batch, W1, b1, W2, b2, W3, b3, Wl, bl):
    src = edge_index[0].astype(jnp.int32)
    dst = edge_index[1].astype(jnp.int32)
    batch = batch.astype(jnp.int32)

    pad = NW * K * CH - E
    dst_p = jnp.concatenate([dst, jnp.full((pad,), DUMMY, jnp.int32)]).reshape(NW, K, CH)
    dst3 = jnp.concatenate([dst_p, jnp.full((NW, 2, CH), DUMMY, jnp.int32)], axis=1)

    pad2 = NS * K2 * CH - E
    comb = jnp.concatenate(
        [src + (dst << 16),
         jnp.full((pad2,), DUMMY << 16, jnp.int32)]).reshape(NS, K2, CH)
    combo = jnp.concatenate(
        [comb, jnp.full((NS, KP2 - K2, CH), DUMMY << 16, jnp.int32)], axis=1)

    deg_part = _sc_deg(dst3).reshape(NC, NPAD)
    y1, dinv = _mm_first(deg_part, x, W1)
    acc1 = _sc_scatter(y1, combo)
    y2 = _mm_mid(acc1, y1, dinv, b1, W2)
    acc2 = _sc_scatter(y2, combo)
    y3 = _mm_mid(acc2, y2, dinv, b2, W3)
    acc3 = _sc_scatter(y3, combo)
    batch2 = jnp.broadcast_to(
        jnp.pad(batch, (0, NBLK * BN - N))[None, :], (8, NBLK * BN))
    return _mm_final(acc3, y3, dinv, b3, batch2, Wl, bl)


# speedup vs baseline: 1.1811x; 1.1811x over previous
import functools

import jax
import jax.numpy as jnp
from jax import lax
from jax.experimental import pallas as pl
from jax.experimental.pallas import tpu as pltpu
from jax.experimental.pallas import tpu_sc as plsc

N = 10000
E = 320000
D = 128
G = 64
DOUT = 10

NC = 2
NS = 16
NW = NC * NS
CH = 64
K = 158
KP = K + 2
K2 = 314
KP2 = K2 + 2
NPAD = 10112
RPT = NPAD // NS
DUMMY = N + 8

HREAL = 5120
HN = 5248
HRPT = HN // NS
LOCDUMMY = HREAL + 8

BN = 2560
NBLK = 4

_mesh = plsc.VectorSubcoreMesh(core_axis_name="c", subcore_axis_name="s")


@functools.partial(
    pl.kernel,
    out_type=jax.ShapeDtypeStruct((NC * NPAD,), jnp.float32),
    mesh=_mesh,
    scratch_types=[
        pltpu.VMEM((KP, CH), jnp.int32),
        pltpu.VMEM((CH,), jnp.float32),
        pltpu.VMEM((RPT + 8, ), jnp.float32),
        pltpu.VMEM_SHARED((NPAD,), jnp.float32),
    ],
)
def _sc_deg(dst_hbm, out_hbm, dst_v, ones_v, zb_v, deg_sh):
    c = lax.axis_index("c")
    s = lax.axis_index("s")
    wid = s * NC + c
    pltpu.sync_copy(dst_hbm.at[wid], dst_v)

    z16 = jnp.zeros((16,), jnp.float32)
    ones16 = jnp.ones((16,), jnp.float32)

    def _fill(i, carry):
        ones_v[pl.ds(i * 16, 16)] = ones16
        return carry

    lax.fori_loop(0, CH // 16, _fill, 0)

    def _zero(i, carry):
        zb_v[pl.ds(i * 16, 16)] = z16
        return carry

    lax.fori_loop(0, (RPT + 8) // 16, _zero, 0)

    base = s * RPT
    pltpu.sync_copy(zb_v.at[pl.ds(0, RPT)], deg_sh.at[pl.ds(base, RPT)])
    plsc.subcore_barrier()

    def _count(j, carry):
        pltpu.sync_copy(ones_v, deg_sh.at[dst_v.at[j]], add=True)
        return carry

    lax.fori_loop(0, KP, _count, 0)
    plsc.subcore_barrier()
    pltpu.sync_copy(deg_sh.at[pl.ds(base, RPT)], zb_v.at[pl.ds(0, RPT)])
    pltpu.sync_copy(zb_v.at[pl.ds(0, RPT)],
                    out_hbm.at[pl.ds(c * NPAD + base, RPT)])


@functools.partial(
    pl.kernel,
    out_type=jax.ShapeDtypeStruct((NC, HN, D), jnp.float32),
    mesh=_mesh,
    scratch_types=[
        pltpu.VMEM((KP2, CH), jnp.int32),
        pltpu.VMEM((2, CH), jnp.int32),
        pltpu.VMEM((2, CH), jnp.int32),
        pltpu.VMEM((2, CH, D), jnp.float32),
        pltpu.VMEM_SHARED((HN, D), jnp.float32),
        (pltpu.SemaphoreType.DMA,) * 2,
    ],
)
def _sc_scatter(yh, combo_hbm, out_hbm, combo_v, srcst, dstst, rows, acc_sh,
                gsem):
    c = lax.axis_index("c")
    s = lax.axis_index("s")
    pltpu.sync_copy(combo_hbm.at[s], combo_v)
    off = c * HREAL

    def _unpack(j, b):
        for i in range(CH // 16):
            v = combo_v[j, pl.ds(i * 16, 16)]
            srcst[b, pl.ds(i * 16, 16)] = v & 0xFFFF
            loc = (v >> 16) - off
            bad = (loc < 0) | (loc >= HREAL)
            dstst[b, pl.ds(i * 16, 16)] = jnp.where(bad, HREAL + (loc & 63),
                                                    loc)

    z16 = jnp.zeros((16,), jnp.float32)

    def _zero(i, carry):
        rows[0, i // (D // 16), pl.ds((i % (D // 16)) * 16, 16)] = z16
        return carry

    lax.fori_loop(0, CH * D // 16, _zero, 0)

    base = s * HRPT
    for k in range(HRPT // CH):
        pltpu.sync_copy(rows.at[0], acc_sh.at[pl.ds(base + k * CH, CH)])
    rem = HRPT % CH
    if rem:
        pltpu.sync_copy(rows.at[0, pl.ds(0, rem)],
                        acc_sh.at[pl.ds(base + (HRPT // CH) * CH, rem)])
    plsc.subcore_barrier()

    def _gather(j, b):
        return pltpu.make_async_copy(yh.at[srcst.at[b]], rows.at[b], gsem[b])

    _unpack(0, 0)
    _gather(0, 0).start()
    _unpack(1, 1)
    _gather(1, 1).start()

    def _body(t, carry):
        j = t * 2
        _gather(j, 0).wait()
        pltpu.sync_copy(rows.at[0], acc_sh.at[dstst.at[0]], add=True)
        _unpack(j + 2, 0)
        _gather(j + 2, 0).start()
        _gather(j + 1, 1).wait()
        pltpu.sync_copy(rows.at[1], acc_sh.at[dstst.at[1]], add=True)
        _unpack(j + 3, 1)
        _gather(j + 3, 1).start()
        return carry

    lax.fori_loop(0, K2 // 2, _body, 0)
    _gather(K2, 0).wait()
    _gather(K2 + 1, 1).wait()
    plsc.subcore_barrier()

    nfull = HRPT // CH
    rem = HRPT % CH
    for k in range(nfull):
        b = k % 2
        pltpu.sync_copy(acc_sh.at[pl.ds(base + k * CH, CH)], rows.at[b])
        pltpu.make_async_copy(rows.at[b], out_hbm.at[c, pl.ds(base + k * CH, CH)],
                              gsem[b]).start()
        if k >= 1:
            pb = (k - 1) % 2
            pltpu.make_async_copy(
                rows.at[pb], out_hbm.at[c, pl.ds(base + (k - 1) * CH, CH)],
                gsem[pb]).wait()
    if rem:
        b = nfull % 2
        pltpu.sync_copy(acc_sh.at[pl.ds(base + nfull * CH, rem)],
                        rows.at[b, pl.ds(0, rem)])
        pltpu.sync_copy(rows.at[b, pl.ds(0, rem)],
                        out_hbm.at[c, pl.ds(base + nfull * CH, rem)])
    pb = (nfull - 1) % 2
    pltpu.make_async_copy(rows.at[pb],
                          out_hbm.at[c, pl.ds(base + (nfull - 1) * CH, CH)],
                          gsem[pb]).wait()


def _acc_spec():
    return pl.BlockSpec((1, BN, D), lambda i: (i // 2, i % 2, 0))


def _mm_first_body(deg_ref, x_ref, w_ref, y_ref, dinv_ref):
    degs = jnp.maximum(deg_ref[0] + deg_ref[1] + 1.0, 1.0)
    dinv = lax.rsqrt(degs)
    dinv_ref[...] = jnp.broadcast_to(dinv[None, :], (8, BN))
    y_ref[...] = dinv[:, None] * jnp.dot(
        x_ref[...], w_ref[...], preferred_element_type=jnp.float32)


def _mm_first(deg_part, x, w):
    return pl.pallas_call(
        _mm_first_body,
        grid=(NBLK,),
        in_specs=[
            pl.BlockSpec((NC, BN), lambda i: (0, i)),
            pl.BlockSpec((BN, D), lambda i: (i, 0)),
            pl.BlockSpec((D, D), lambda i: (0, 0)),
        ],
        out_specs=[
            pl.BlockSpec((BN, D), lambda i: (i, 0)),
            pl.BlockSpec((8, BN), lambda i: (0, i)),
        ],
        out_shape=[
            jax.ShapeDtypeStruct((N, D), jnp.float32),
            jax.ShapeDtypeStruct((8, NBLK * BN), jnp.float32),
        ],
    )(deg_part, x, w)


def _mm_mid_body(acc_ref, y_ref, dinv_ref, b_ref, w_ref, out_ref):
    a = acc_ref[0] + y_ref[...]
    dv = dinv_ref[0]
    h = jnp.maximum(dv[:, None] * a + b_ref[...][None, :], 0.0)
    out_ref[...] = dv[:, None] * jnp.dot(
        h, w_ref[...], preferred_element_type=jnp.float32)


def _mm_mid(acc, y, dinv, b, w):
    return pl.pallas_call(
        _mm_mid_body,
        grid=(NBLK,),
        in_specs=[
            _acc_spec(),
            pl.BlockSpec((BN, D), lambda i: (i, 0)),
            pl.BlockSpec((8, BN), lambda i: (0, i)),
            pl.BlockSpec((D,), lambda i: (0,)),
            pl.BlockSpec((D, D), lambda i: (0, 0)),
        ],
        out_specs=pl.BlockSpec((BN, D), lambda i: (i, 0)),
        out_shape=jax.ShapeDtypeStruct((N, D), jnp.float32),
    )(acc, y, dinv, b, w)


def _mm_final_body(acc_ref, y_ref, dinv_ref, b_ref, batch_ref, wl_ref, bl_ref,
                   sums_ref, counts_ref, out_ref):
    i = pl.program_id(0)

    @pl.when(i == 0)
    def _():
        sums_ref[...] = jnp.zeros_like(sums_ref)
        counts_ref[...] = jnp.zeros_like(counts_ref)

    a = acc_ref[0] + y_ref[...]
    h = dinv_ref[0][:, None] * a + b_ref[...][None, :]
    validc = (i * BN + lax.broadcasted_iota(jnp.int32, (BN, 1), 0)) < N
    h = jnp.where(validc, h, 0.0)
    validr = (i * BN + lax.broadcasted_iota(jnp.int32, (1, BN), 1)) < N
    oh = jnp.where(
        (lax.broadcasted_iota(jnp.int32, (G, BN), 0) == batch_ref[0][None, :])
        & validr, 1.0, 0.0)
    sums_ref[...] += jnp.dot(oh, h, preferred_element_type=jnp.float32)
    counts_ref[...] += jnp.sum(oh, axis=1, keepdims=True)

    @pl.when(i == NBLK - 1)
    def _():
        g = sums_ref[...] / jnp.maximum(counts_ref[...], 1.0)
        logits = jnp.dot(g, wl_ref[...],
                         preferred_element_type=jnp.float32) + bl_ref[...][None, :]
        m = jnp.max(logits, axis=1, keepdims=True)
        lse = jnp.log(jnp.sum(jnp.exp(logits - m), axis=1, keepdims=True)) + m
        out_ref[...] = logits - lse


def _mm_final(acc, y, dinv, b, batch, wl, bl):
    _, _, out = pl.pallas_call(
        _mm_final_body,
        grid=(NBLK,),
        in_specs=[
            _acc_spec(),
            pl.BlockSpec((BN, D), lambda i: (i, 0)),
            pl.BlockSpec((8, BN), lambda i: (0, i)),
            pl.BlockSpec((D,), lambda i: (0,)),
            pl.BlockSpec((8, BN), lambda i: (0, i)),
            pl.BlockSpec((D, DOUT), lambda i: (0, 0)),
            pl.BlockSpec((DOUT,), lambda i: (0,)),
        ],
        out_specs=[
            pl.BlockSpec((G, D), lambda i: (0, 0)),
            pl.BlockSpec((G, 1), lambda i: (0, 0)),
            pl.BlockSpec((G, DOUT), lambda i: (0, 0)),
        ],
        out_shape=[
            jax.ShapeDtypeStruct((G, D), jnp.float32),
            jax.ShapeDtypeStruct((G, 1), jnp.float32),
            jax.ShapeDtypeStruct((G, DOUT), jnp.float32),
        ],
    )(acc, y, dinv, b, batch, wl, bl)
    return out


def kernel(x, edge_index, batch, W1, b1, W2, b2, W3, b3, Wl, bl):
    src = edge_index[0].astype(jnp.int32)
    dst = edge_index[1].astype(jnp.int32)
    batch = batch.astype(jnp.int32)

    pad = NW * K * CH - E
    dst_p = jnp.concatenate([dst, jnp.full((pad,), DUMMY, jnp.int32)]).reshape(NW, K, CH)
    dst3 = jnp.concatenate([dst_p, jnp.full((NW, 2, CH), DUMMY, jnp.int32)], axis=1)

    pad2 = NS * K2 * CH - E
    comb = jnp.concatenate(
        [src + (dst << 16),
         jnp.full((pad2,), DUMMY << 16, jnp.int32)]).reshape(NS, K2, CH)
    combo = jnp.concatenate(
        [comb, jnp.full((NS, KP2 - K2, CH), DUMMY << 16, jnp.int32)], axis=1)

    deg_part = _sc_deg(dst3).reshape(NC, NPAD)
    y1, dinv = _mm_first(deg_part, x, W1)
    acc1 = _sc_scatter(y1, combo)
    y2 = _mm_mid(acc1, y1, dinv, b1, W2)
    acc2 = _sc_scatter(y2, combo)
    y3 = _mm_mid(acc2, y2, dinv, b2, W3)
    acc3 = _sc_scatter(y3, combo)
    batch2 = jnp.broadcast_to(
        jnp.pad(batch, (0, NBLK * BN - N))[None, :], (8, NBLK * BN))
    return _mm_final(acc3, y3, dinv, b3, batch2, Wl, bl)


# speedup vs baseline: 1.4153x; 1.1984x over previous
import functools

import jax
import jax.numpy as jnp
from jax import lax
from jax.experimental import pallas as pl
from jax.experimental.pallas import tpu as pltpu
from jax.experimental.pallas import tpu_sc as plsc

N = 10000
E = 320000
D = 128
G = 64
DOUT = 10

NC = 2
NS = 16
NW = NC * NS
CH = 80
K = 126
KP = K + 2
K2 = 250
KP2 = K2 + 2
NPAD = 10112
RPT = NPAD // NS
DUMMY = N + 8

HREAL = 5120
HN = 5248
HRPT = HN // NS
LOCDUMMY = HREAL + 8

BN = 2560
NBLK = 4

_mesh = plsc.VectorSubcoreMesh(core_axis_name="c", subcore_axis_name="s")


@functools.partial(
    pl.kernel,
    out_type=jax.ShapeDtypeStruct((NC * NPAD,), jnp.float32),
    mesh=_mesh,
    scratch_types=[
        pltpu.VMEM((KP, CH), jnp.int32),
        pltpu.VMEM((CH,), jnp.float32),
        pltpu.VMEM((RPT + 8, ), jnp.float32),
        pltpu.VMEM_SHARED((NPAD,), jnp.float32),
    ],
)
def _sc_deg(dst_hbm, out_hbm, dst_v, ones_v, zb_v, deg_sh):
    c = lax.axis_index("c")
    s = lax.axis_index("s")
    wid = s * NC + c
    pltpu.sync_copy(dst_hbm.at[wid], dst_v)

    z16 = jnp.zeros((16,), jnp.float32)
    ones16 = jnp.ones((16,), jnp.float32)

    def _fill(i, carry):
        ones_v[pl.ds(i * 16, 16)] = ones16
        return carry

    lax.fori_loop(0, CH // 16, _fill, 0)

    def _zero(i, carry):
        zb_v[pl.ds(i * 16, 16)] = z16
        return carry

    lax.fori_loop(0, (RPT + 8) // 16, _zero, 0)

    base = s * RPT
    pltpu.sync_copy(zb_v.at[pl.ds(0, RPT)], deg_sh.at[pl.ds(base, RPT)])
    plsc.subcore_barrier()

    def _count(j, carry):
        pltpu.sync_copy(ones_v, deg_sh.at[dst_v.at[j]], add=True)
        return carry

    lax.fori_loop(0, KP, _count, 0)
    plsc.subcore_barrier()
    pltpu.sync_copy(deg_sh.at[pl.ds(base, RPT)], zb_v.at[pl.ds(0, RPT)])
    pltpu.sync_copy(zb_v.at[pl.ds(0, RPT)],
                    out_hbm.at[pl.ds(c * NPAD + base, RPT)])


@functools.partial(
    pl.kernel,
    out_type=jax.ShapeDtypeStruct((NC, HN, D), jnp.float32),
    mesh=_mesh,
    scratch_types=[
        pltpu.VMEM((KP2, CH), jnp.int32),
        pltpu.VMEM((KP2, CH), jnp.int32),
        pltpu.VMEM((2, CH, D), jnp.float32),
        pltpu.VMEM_SHARED((HN, D), jnp.float32),
        (pltpu.SemaphoreType.DMA,) * 2,
    ],
)
def _sc_scatter(yh, src_hbm, dst_hbm, out_hbm, src_v, dst_v, rows, acc_sh,
                gsem):
    c = lax.axis_index("c")
    s = lax.axis_index("s")
    pltpu.sync_copy(src_hbm.at[s], src_v)
    pltpu.sync_copy(dst_hbm.at[s], dst_v)

    off = c * HREAL
    nsub = CH // 16

    def _remap(i, carry):
        v = dst_v[i // nsub, pl.ds((i % nsub) * 16, 16)]
        loc = v - off
        bad = (loc < 0) | (loc >= HREAL)
        dst_v[i // nsub, pl.ds((i % nsub) * 16, 16)] = jnp.where(
            bad, HREAL + (loc & 63), loc)
        return carry

    lax.fori_loop(0, KP2 * nsub, _remap, 0)

    z16 = jnp.zeros((16,), jnp.float32)

    def _zero(i, carry):
        rows[0, i // (D // 16), pl.ds((i % (D // 16)) * 16, 16)] = z16
        return carry

    lax.fori_loop(0, CH * D // 16, _zero, 0)

    base = s * HRPT
    for k in range(HRPT // CH):
        pltpu.sync_copy(rows.at[0], acc_sh.at[pl.ds(base + k * CH, CH)])
    rem = HRPT % CH
    if rem:
        pltpu.sync_copy(rows.at[0, pl.ds(0, rem)],
                        acc_sh.at[pl.ds(base + (HRPT // CH) * CH, rem)])
    plsc.subcore_barrier()

    def _gather(j, b):
        return pltpu.make_async_copy(yh.at[src_v.at[j]], rows.at[b], gsem[b])

    _gather(0, 0).start()
    _gather(1, 1).start()

    def _body(t, carry):
        j = t * 2
        _gather(j, 0).wait()
        pltpu.sync_copy(rows.at[0], acc_sh.at[dst_v.at[j]], add=True)
        _gather(j + 2, 0).start()
        _gather(j + 1, 1).wait()
        pltpu.sync_copy(rows.at[1], acc_sh.at[dst_v.at[j + 1]], add=True)
        _gather(j + 3, 1).start()
        return carry

    lax.fori_loop(0, K2 // 2, _body, 0)
    _gather(K2, 0).wait()
    _gather(K2 + 1, 1).wait()
    plsc.subcore_barrier()

    nfull = HRPT // CH
    rem = HRPT % CH
    for k in range(nfull):
        b = k % 2
        pltpu.sync_copy(acc_sh.at[pl.ds(base + k * CH, CH)], rows.at[b])
        pltpu.make_async_copy(rows.at[b], out_hbm.at[c, pl.ds(base + k * CH, CH)],
                              gsem[b]).start()
        if k >= 1:
            pb = (k - 1) % 2
            pltpu.make_async_copy(
                rows.at[pb], out_hbm.at[c, pl.ds(base + (k - 1) * CH, CH)],
                gsem[pb]).wait()
    if rem:
        b = nfull % 2
        pltpu.sync_copy(acc_sh.at[pl.ds(base + nfull * CH, rem)],
                        rows.at[b, pl.ds(0, rem)])
        pltpu.sync_copy(rows.at[b, pl.ds(0, rem)],
                        out_hbm.at[c, pl.ds(base + nfull * CH, rem)])
    pb = (nfull - 1) % 2
    pltpu.make_async_copy(rows.at[pb],
                          out_hbm.at[c, pl.ds(base + (nfull - 1) * CH, CH)],
                          gsem[pb]).wait()


def _acc_spec():
    return pl.BlockSpec((1, BN, D), lambda i: (i // 2, i % 2, 0))


def _mm_first_body(deg_ref, x_ref, w_ref, y_ref, dinv_ref):
    degs = jnp.maximum(deg_ref[0] + deg_ref[1] + 1.0, 1.0)
    dinv = lax.rsqrt(degs)
    dinv_ref[...] = jnp.broadcast_to(dinv[None, :], (8, BN))
    y_ref[...] = dinv[:, None] * jnp.dot(
        x_ref[...], w_ref[...], preferred_element_type=jnp.float32)


def _mm_first(deg_part, x, w):
    return pl.pallas_call(
        _mm_first_body,
        grid=(NBLK,),
        in_specs=[
            pl.BlockSpec((NC, BN), lambda i: (0, i)),
            pl.BlockSpec((BN, D), lambda i: (i, 0)),
            pl.BlockSpec((D, D), lambda i: (0, 0)),
        ],
        out_specs=[
            pl.BlockSpec((BN, D), lambda i: (i, 0)),
            pl.BlockSpec((8, BN), lambda i: (0, i)),
        ],
        out_shape=[
            jax.ShapeDtypeStruct((N, D), jnp.float32),
            jax.ShapeDtypeStruct((8, NBLK * BN), jnp.float32),
        ],
    )(deg_part, x, w)


def _mm_mid_body(acc_ref, y_ref, dinv_ref, b_ref, w_ref, out_ref):
    a = acc_ref[0] + y_ref[...]
    dv = dinv_ref[0]
    h = jnp.maximum(dv[:, None] * a + b_ref[...][None, :], 0.0)
    out_ref[...] = dv[:, None] * jnp.dot(
        h, w_ref[...], preferred_element_type=jnp.float32)


def _mm_mid(acc, y, dinv, b, w):
    return pl.pallas_call(
        _mm_mid_body,
        grid=(NBLK,),
        in_specs=[
            _acc_spec(),
            pl.BlockSpec((BN, D), lambda i: (i, 0)),
            pl.BlockSpec((8, BN), lambda i: (0, i)),
            pl.BlockSpec((D,), lambda i: (0,)),
            pl.BlockSpec((D, D), lambda i: (0, 0)),
        ],
        out_specs=pl.BlockSpec((BN, D), lambda i: (i, 0)),
        out_shape=jax.ShapeDtypeStruct((N, D), jnp.float32),
    )(acc, y, dinv, b, w)


def _mm_final_body(acc_ref, y_ref, dinv_ref, b_ref, batch_ref, wl_ref, bl_ref,
                   sums_ref, counts_ref, out_ref):
    i = pl.program_id(0)

    @pl.when(i == 0)
    def _():
        sums_ref[...] = jnp.zeros_like(sums_ref)
        counts_ref[...] = jnp.zeros_like(counts_ref)

    a = acc_ref[0] + y_ref[...]
    h = dinv_ref[0][:, None] * a + b_ref[...][None, :]
    validc = (i * BN + lax.broadcasted_iota(jnp.int32, (BN, 1), 0)) < N
    h = jnp.where(validc, h, 0.0)
    validr = (i * BN + lax.broadcasted_iota(jnp.int32, (1, BN), 1)) < N
    oh = jnp.where(
        (lax.broadcasted_iota(jnp.int32, (G, BN), 0) == batch_ref[0][None, :])
        & validr, 1.0, 0.0)
    sums_ref[...] += jnp.dot(oh, h, preferred_element_type=jnp.float32)
    counts_ref[...] += jnp.sum(oh, axis=1, keepdims=True)

    @pl.when(i == NBLK - 1)
    def _():
        g = sums_ref[...] / jnp.maximum(counts_ref[...], 1.0)
        logits = jnp.dot(g, wl_ref[...],
                         preferred_element_type=jnp.float32) + bl_ref[...][None, :]
        m = jnp.max(logits, axis=1, keepdims=True)
        lse = jnp.log(jnp.sum(jnp.exp(logits - m), axis=1, keepdims=True)) + m
        out_ref[...] = logits - lse


def _mm_final(acc, y, dinv, b, batch, wl, bl):
    _, _, out = pl.pallas_call(
        _mm_final_body,
        grid=(NBLK,),
        in_specs=[
            _acc_spec(),
            pl.BlockSpec((BN, D), lambda i: (i, 0)),
            pl.BlockSpec((8, BN), lambda i: (0, i)),
            pl.BlockSpec((D,), lambda i: (0,)),
            pl.BlockSpec((8, BN), lambda i: (0, i)),
            pl.BlockSpec((D, DOUT), lambda i: (0, 0)),
            pl.BlockSpec((DOUT,), lambda i: (0,)),
        ],
        out_specs=[
            pl.BlockSpec((G, D), lambda i: (0, 0)),
            pl.BlockSpec((G, 1), lambda i: (0, 0)),
            pl.BlockSpec((G, DOUT), lambda i: (0, 0)),
        ],
        out_shape=[
            jax.ShapeDtypeStruct((G, D), jnp.float32),
            jax.ShapeDtypeStruct((G, 1), jnp.float32),
            jax.ShapeDtypeStruct((G, DOUT), jnp.float32),
        ],
    )(acc, y, dinv, b, batch, wl, bl)
    return out


def kernel(x, edge_index, batch, W1, b1, W2, b2, W3, b3, Wl, bl):
    src = edge_index[0].astype(jnp.int32)
    dst = edge_index[1].astype(jnp.int32)
    batch = batch.astype(jnp.int32)

    pad = NW * K * CH - E
    dst_p = jnp.concatenate([dst, jnp.full((pad,), DUMMY, jnp.int32)]).reshape(NW, K, CH)
    dst3 = jnp.concatenate([dst_p, jnp.full((NW, 2, CH), DUMMY, jnp.int32)], axis=1)

    pad2 = NS * K2 * CH - E
    src_q = jnp.concatenate([src, jnp.zeros((pad2,), jnp.int32)]).reshape(NS, K2, CH)
    dst_q = jnp.concatenate([dst, jnp.full((pad2,), DUMMY, jnp.int32)]).reshape(NS, K2, CH)
    src2 = jnp.concatenate([src_q, jnp.zeros((NS, KP2 - K2, CH), jnp.int32)], axis=1)
    dst2 = jnp.concatenate([dst_q, jnp.full((NS, KP2 - K2, CH), DUMMY, jnp.int32)], axis=1)

    deg_part = _sc_deg(dst3).reshape(NC, NPAD)
    y1, dinv = _mm_first(deg_part, x, W1)
    acc1 = _sc_scatter(y1, src2, dst2)
    y2 = _mm_mid(acc1, y1, dinv, b1, W2)
    acc2 = _sc_scatter(y2, src2, dst2)
    y3 = _mm_mid(acc2, y2, dinv, b2, W3)
    acc3 = _sc_scatter(y3, src2, dst2)
    batch2 = jnp.broadcast_to(
        jnp.pad(batch, (0, NBLK * BN - N))[None, :], (8, NBLK * BN))
    return _mm_final(acc3, y3, dinv, b3, batch2, Wl, bl)
